# baseline (speedup 1.0000x reference)
import functools

import jax
import jax.numpy as jnp
import numpy as np
from jax import lax
from jax.experimental import pallas as pl
from jax.experimental.pallas import tpu as pltpu
from jax.experimental.pallas import tpu_sc as plsc

_B = 32
_C = 384
_G = 96
_GROUPBY = 4
_NSEL = 47

_B_TC = 24
_B_SC = _B - _B_TC

_LT = _C // 128
_ROWS_PER_B = 9408
_UNIT = 24
_NWORK = 32
_WPB = _NWORK // _B_SC
_ROWS_PER_W = _ROWS_PER_B // _WPB
_CHUNK_ROWS = 168
_NCHUNK_W = _ROWS_PER_W // _CHUNK_ROWS


def _group_mask():
    key = jax.random.key(42)
    keys = jax.random.split(key, _B)
    notp = jnp.arange(1, _G, dtype=jnp.int32)
    chosen = jax.vmap(lambda k: jax.random.permutation(k, notp)[:_NSEL])(keys)
    mask = jnp.zeros((_B, _G), jnp.float32).at[:, 0].set(1.0)
    mask = mask.at[jnp.arange(_B)[:, None], chosen].set(1.0)
    return mask


_MASK_BC = np.asarray(
    jax.device_get(jnp.repeat(_group_mask(), _GROUPBY, axis=1))
)


def _sc_body(x_hbm, m_hbm, o_hbm, buf, mbuf, gsem, ssem):
    info = plsc.get_sparse_core_info()
    nc = info.num_cores
    w = lax.axis_index("s") * nc + lax.axis_index("c")
    pltpu.sync_copy(m_hbm.at[_B_TC + w // _WPB], mbuf)
    in_base = _B_TC * _ROWS_PER_B + w * _ROWS_PER_W
    out_base = w * _ROWS_PER_W

    def gather(ci, s):
        return pltpu.make_async_copy(
            x_hbm.at[pl.ds(in_base + ci * _CHUNK_ROWS, _CHUNK_ROWS)],
            buf.at[s], gsem.at[s])

    def scatter(ci, s):
        return pltpu.make_async_copy(
            buf.at[s],
            o_hbm.at[pl.ds(out_base + ci * _CHUNK_ROWS, _CHUNK_ROWS)],
            ssem.at[s])

    def compute(s):
        for lt in range(_LT):
            for j in range(128 // 16):
                mv = mbuf[pl.ds(lt * 128 + j * 16, 16)]

                def unit(u, mv):
                    for wi in range(8):
                        r = u * _UNIT + lt * 8 + wi
                        sl = (s, r, pl.ds(j * 16, 16))
                        buf[sl] = buf[sl] * mv
                    return mv

                lax.fori_loop(0, _CHUNK_ROWS // _UNIT, unit, mv)

    gather(0, 0).start()
    gather(1, 1).start()

    def pair(p, carry):
        a = 2 * p
        gather(a, 0).wait()
        compute(0)
        scatter(a, 0).start()

        @pl.when(p < _NCHUNK_W // 2 - 1)
        def _():
            scatter(a, 0).wait()
            gather(a + 2, 0).start()

        gather(a + 1, 1).wait()
        compute(1)
        scatter(a + 1, 1).start()

        @pl.when(p < _NCHUNK_W // 2 - 1)
        def _():
            scatter(a + 1, 1).wait()
            gather(a + 3, 1).start()

        return carry

    lax.fori_loop(0, _NCHUNK_W // 2, pair, 0)
    scatter(_NCHUNK_W - 2, 0).wait()
    scatter(_NCHUNK_W - 1, 1).wait()


def _mul_body(x_ref, m_ref, o_ref):
    o_ref[...] = x_ref[...] * m_ref[...]


def kernel(input):
    B, C, H, W = input.shape
    hw = H * W
    m = jnp.asarray(_MASK_BC)

    xt = jnp.transpose(input, (0, 2, 3, 1)).reshape(B, hw, C)
    bb = 2
    out_tc = pl.pallas_call(
        _mul_body,
        grid=(_B_TC // bb,),
        in_specs=[
            pl.BlockSpec((bb, hw, C), lambda b: (b, 0, 0)),
            pl.BlockSpec((bb, 1, C), lambda b: (b, 0, 0)),
        ],
        out_specs=pl.BlockSpec((bb, hw, C), lambda b: (b, 0, 0)),
        out_shape=jax.ShapeDtypeStruct((_B_TC, hw, C), jnp.float32),
    )(xt, m.reshape(B, 1, C))

    rows = B * H * (W // 8) * _LT * 8
    x2 = (
        input.reshape(B, _LT, 128, H, W // 8, 8)
        .transpose(0, 3, 4, 1, 5, 2)
        .reshape(rows, 128)
    )
    mesh = plsc.VectorSubcoreMesh(core_axis_name="c", subcore_axis_name="s")
    out_sc = functools.partial(
        pl.kernel,
        mesh=mesh,
        out_type=jax.ShapeDtypeStruct((_B_SC * _ROWS_PER_B, 128), jnp.float32),
        scratch_types=[
            pltpu.VMEM((2, _CHUNK_ROWS, 128), jnp.float32),
            pltpu.VMEM((_C,), jnp.float32),
            pltpu.SemaphoreType.DMA((2,)),
            pltpu.SemaphoreType.DMA((2,)),
        ],
    )(_sc_body)(x2, m)

    a4 = jnp.transpose(out_tc.reshape(_B_TC, H, W, C), (0, 3, 1, 2))
    b4 = (
        out_sc.reshape(_B_SC, H, W // 8, _LT, 8, 128)
        .transpose(0, 3, 5, 1, 2, 4)
        .reshape(_B_SC, C, H, W)
    )
    return jnp.concatenate([a4, b4], axis=0)


# speedup vs baseline: 1.6249x; 1.6249x over previous
import functools

import jax
import jax.numpy as jnp
import numpy as np
from jax import lax
from jax.experimental import pallas as pl
from jax.experimental.pallas import tpu as pltpu
from jax.experimental.pallas import tpu_sc as plsc

_B = 32
_C = 384
_G = 96
_GROUPBY = 4
_NSEL = 47

_LT = _C // 128
_ROWS_PER_B = 9408
_UNIT = 24
_CHUNK_ROWS = 192
_NCHUNK = _ROWS_PER_B // _CHUNK_ROWS
_NSLOT = 3


def _group_mask():
    key = jax.random.key(42)
    keys = jax.random.split(key, _B)
    notp = jnp.arange(1, _G, dtype=jnp.int32)
    chosen = jax.vmap(lambda k: jax.random.permutation(k, notp)[:_NSEL])(keys)
    mask = jnp.zeros((_B, _G), jnp.float32).at[:, 0].set(1.0)
    mask = mask.at[jnp.arange(_B)[:, None], chosen].set(1.0)
    return mask


_MASK_BC = np.asarray(
    jax.device_get(jnp.repeat(_group_mask(), _GROUPBY, axis=1))
)


def _sc_body(x_hbm, m_hbm, o_hbm, buf, mbuf, gsem, ssem):
    info = plsc.get_sparse_core_info()
    nc = info.num_cores
    b = lax.axis_index("s") * nc + lax.axis_index("c")
    pltpu.sync_copy(m_hbm.at[b], mbuf)
    base = b * _ROWS_PER_B

    def gather(ci, s):
        return pltpu.make_async_copy(
            x_hbm.at[pl.ds(base + ci * _CHUNK_ROWS, _CHUNK_ROWS)],
            buf.at[s], gsem.at[s])

    def scatter(ci, s):
        return pltpu.make_async_copy(
            buf.at[s],
            o_hbm.at[pl.ds(base + ci * _CHUNK_ROWS, _CHUNK_ROWS)],
            ssem.at[s])

    def compute(s):
        for lt in range(_LT):
            for j in range(128 // 16):
                mv = mbuf[pl.ds(lt * 128 + j * 16, 16)]

                def unit(u, mv):
                    for wi in range(8):
                        r = u * _UNIT + lt * 8 + wi
                        sl = (s, r, pl.ds(j * 16, 16))
                        buf[sl] = buf[sl] * mv
                    return mv

                lax.fori_loop(0, _CHUNK_ROWS // _UNIT, unit, mv)

    def step(ci, carry):
        s = lax.rem(ci, _NSLOT)
        sp = lax.rem(ci + _NSLOT - 1, _NSLOT)

        @pl.when(jnp.logical_and(ci >= _NSLOT, ci < _NCHUNK))
        def _():
            scatter(ci - _NSLOT, s).wait()

        @pl.when(ci < _NCHUNK)
        def _():
            gather(ci, s).start()

        @pl.when(jnp.logical_and(ci >= 1, ci <= _NCHUNK))
        def _():
            gather(ci - 1, sp).wait()
            compute(sp)
            scatter(ci - 1, sp).start()

        return carry

    lax.fori_loop(0, _NCHUNK + 1, step, 0)
    for k in range(_NSLOT):
        ci = _NCHUNK - _NSLOT + k
        scatter(ci, ci % _NSLOT).wait()


def kernel(input):
    B, C, H, W = input.shape
    rows = B * H * (W // 8) * _LT * 8
    x2 = (
        input.reshape(B, _LT, 128, H, W // 8, 8)
        .transpose(0, 3, 4, 1, 5, 2)
        .reshape(rows, 128)
    )
    m = jnp.asarray(_MASK_BC)
    mesh = plsc.VectorSubcoreMesh(core_axis_name="c", subcore_axis_name="s")
    sc_call = functools.partial(
        pl.kernel,
        mesh=mesh,
        out_type=jax.ShapeDtypeStruct((rows, 128), jnp.float32),
        scratch_types=[
            pltpu.VMEM((_NSLOT, _CHUNK_ROWS, 128), jnp.float32),
            pltpu.VMEM((_C,), jnp.float32),
            pltpu.SemaphoreType.DMA((_NSLOT,)),
            pltpu.SemaphoreType.DMA((_NSLOT,)),
        ],
    )(_sc_body)
    out = sc_call(x2, m)
    return (
        out.reshape(B, H, W // 8, _LT, 8, 128)
        .transpose(0, 3, 5, 1, 2, 4)
        .reshape(B, C, H, W)
    )


# speedup vs baseline: 2.1962x; 1.3516x over previous
import jax
import jax.numpy as jnp
import numpy as np
from jax.experimental import pallas as pl

_B = 32
_C = 384
_G = 96
_GROUPBY = 4
_NSEL = 47


def _group_mask():
    key = jax.random.key(42)
    keys = jax.random.split(key, _B)
    notp = jnp.arange(1, _G, dtype=jnp.int32)
    chosen = jax.vmap(lambda k: jax.random.permutation(k, notp)[:_NSEL])(keys)
    mask = jnp.zeros((_B, _G), jnp.float32).at[:, 0].set(1.0)
    mask = mask.at[jnp.arange(_B)[:, None], chosen].set(1.0)
    return mask


_MASK_BC = np.asarray(
    jax.device_get(jnp.repeat(_group_mask(), _GROUPBY, axis=1))
).reshape(_B, 1, _C)


def _mul_body(x_ref, m_ref, o_ref):
    o_ref[...] = x_ref[...] * m_ref[...]


def kernel(input):
    B, C, H, W = input.shape
    hw = H * W
    xt = jnp.transpose(input, (0, 2, 3, 1)).reshape(B, hw, C)
    m = jnp.asarray(_MASK_BC)
    bb = 2
    out = pl.pallas_call(
        _mul_body,
        grid=(B // bb,),
        in_specs=[
            pl.BlockSpec((bb, hw, C), lambda b: (b, 0, 0)),
            pl.BlockSpec((bb, 1, C), lambda b: (b, 0, 0)),
        ],
        out_specs=pl.BlockSpec((bb, hw, C), lambda b: (b, 0, 0)),
        out_shape=jax.ShapeDtypeStruct((B, hw, C), jnp.float32),
    )(xt, m)
    return jnp.transpose(out.reshape(B, H, W, C), (0, 3, 1, 2))
